# baseline (speedup 1.0000x reference)
import functools

import jax
import jax.numpy as jnp
from jax.experimental import pallas as pl
from jax.experimental.pallas import tpu as pltpu


def _row_lse_kernel(emb_ref, lse_ref):
    x = emb_ref[...]
    m = jnp.max(x, axis=-1, keepdims=True)
    lse = jnp.log(jnp.sum(jnp.exp(x - m), axis=-1, keepdims=True)) + m
    lse_ref[...] = jnp.broadcast_to(lse, lse_ref.shape).astype(jnp.bfloat16)


def _fused_tile(tok_ref, tgt_ref, emb_ref, lse_ref, logits_ref, part_ref, *,
                n_rows):
    tr, v = logits_ref.shape
    tok = tok_ref[0]
    lane = jax.lax.broadcasted_iota(jnp.int32, (tr, v), 1)
    sel = (lane == tok).astype(jnp.bfloat16)
    x = jnp.dot(sel, emb_ref[...],
                preferred_element_type=jnp.float32)
    logits_ref[...] = x

    lse_tok = jnp.dot(sel, lse_ref[...],
                      preferred_element_type=jnp.float32)[:, :1]

    tgt = tgt_ref[0]
    picked = jnp.sum(jnp.where(lane == tgt, x, 0.0),
                     axis=-1, keepdims=True)
    per_row = lse_tok - picked

    row0 = pl.program_id(0) * tr
    live = (row0 + jax.lax.broadcasted_iota(jnp.int32, (tr, 1), 0)) < n_rows
    tile_sum = jnp.sum(jnp.where(live, per_row, 0.0))
    part_ref[...] = jnp.full(part_ref.shape, tile_sum, jnp.float32)


def kernel(idx, emb, targets, *, row_tile=256):
    B, T = idx.shape
    V = emb.shape[0]
    N = B * T
    assert V % 128 == 0, "vocab assumed lane-aligned"

    tr = min(row_tile, N)
    n_tiles = -(-N // tr)
    Np = n_tiles * tr

    tok = idx.reshape(N).astype(jnp.int32)
    tgt = targets.reshape(N).astype(jnp.int32)
    if Np != N:
        tok = jnp.pad(tok, (0, Np - N))
        tgt = jnp.pad(tgt, (0, Np - N))
    tok3 = tok.reshape(n_tiles, tr, 1)
    tgt3 = tgt.reshape(n_tiles, tr, 1)
    emb_bf = emb.astype(jnp.bfloat16)

    lse_rows = min(512, V)
    lse_mat = pl.pallas_call(
        _row_lse_kernel,
        out_shape=jax.ShapeDtypeStruct((V, 128), jnp.bfloat16),
        grid=(V // lse_rows,),
        in_specs=[pl.BlockSpec((lse_rows, V), lambda i: (i, 0))],
        out_specs=pl.BlockSpec((lse_rows, 128), lambda i: (i, 0)),
        compiler_params=pltpu.CompilerParams(
            dimension_semantics=("parallel",)),
    )(emb)

    logits, parts = pl.pallas_call(
        functools.partial(_fused_tile, n_rows=N),
        out_shape=(jax.ShapeDtypeStruct((Np, V), jnp.float32),
                   jax.ShapeDtypeStruct((n_tiles, 8, 128), jnp.float32)),
        grid=(n_tiles,),
        in_specs=[pl.BlockSpec((1, tr, 1), lambda i: (i, 0, 0)),
                  pl.BlockSpec((1, tr, 1), lambda i: (i, 0, 0)),
                  pl.BlockSpec((V, V), lambda i: (0, 0)),
                  pl.BlockSpec((V, 128), lambda i: (0, 0))],
        out_specs=(pl.BlockSpec((tr, V), lambda i: (i, 0)),
                   pl.BlockSpec((1, 8, 128), lambda i: (i, 0, 0))),
        compiler_params=pltpu.CompilerParams(
            dimension_semantics=("parallel",),
            vmem_limit_bytes=60 * 1024 * 1024),
    )(tok3, tgt3, emb_bf, lse_mat)

    loss = jnp.sum(parts[:, 0, 0]) / N
    return logits[:N], loss


# speedup vs baseline: 1.3941x; 1.3941x over previous
import functools

import jax
import jax.numpy as jnp
import numpy as np
from jax.experimental import pallas as pl
from jax.experimental.pallas import tpu as pltpu
from jax.sharding import Mesh, PartitionSpec as P


def _row_lse_kernel(emb_ref, lse_ref):
    x = emb_ref[...]
    m = jnp.max(x, axis=-1, keepdims=True)
    lse = jnp.log(jnp.sum(jnp.exp(x - m), axis=-1, keepdims=True)) + m
    lse_ref[...] = jnp.broadcast_to(lse, lse_ref.shape).astype(jnp.bfloat16)


def _fused_tile(tok_ref, tgt_ref, emb_ref, lse_ref, logits_ref, part_ref, *,
                n_rows):
    tr, v = logits_ref.shape
    tok = tok_ref[0]
    lane = jax.lax.broadcasted_iota(jnp.int32, (tr, v), 1)
    sel = (lane == tok).astype(jnp.bfloat16)
    x = jnp.dot(sel, emb_ref[...],
                preferred_element_type=jnp.float32)
    logits_ref[...] = x

    lse_tok = jnp.dot(sel, lse_ref[...],
                      preferred_element_type=jnp.float32)[:, :1]

    tgt = tgt_ref[0]
    picked = jnp.sum(jnp.where(lane == tgt, x, 0.0),
                     axis=-1, keepdims=True)
    per_row = lse_tok - picked

    row0 = pl.program_id(0) * tr
    live = (row0 + jax.lax.broadcasted_iota(jnp.int32, (tr, 1), 0)) < n_rows
    tile_sum = jnp.sum(jnp.where(live, per_row, 0.0))
    part_ref[...] = jnp.full(part_ref.shape, tile_sum, jnp.float32)


def _row_lse(emb):
    v = emb.shape[0]
    lse_rows = min(512, v)
    return pl.pallas_call(
        _row_lse_kernel,
        out_shape=jax.ShapeDtypeStruct((v, 128), jnp.bfloat16),
        grid=(v // lse_rows,),
        in_specs=[pl.BlockSpec((lse_rows, v), lambda i: (i, 0))],
        out_specs=pl.BlockSpec((lse_rows, 128), lambda i: (i, 0)),
        compiler_params=pltpu.CompilerParams(
            dimension_semantics=("parallel",)),
    )(emb)


def _shard_body(tok3, tgt3, emb, *, tr, n_tiles, n_rows):
    return _tiles_call(tok3, tgt3, emb.astype(jnp.bfloat16), _row_lse(emb),
                       tr=tr, n_tiles=n_tiles, n_rows=n_rows)


def _tiles_call(tok3, tgt3, emb_bf, lse_mat, *, tr, n_tiles, n_rows):
    v = emb_bf.shape[0]
    return pl.pallas_call(
        functools.partial(_fused_tile, n_rows=n_rows),
        out_shape=(jax.ShapeDtypeStruct((n_tiles * tr, v), jnp.float32),
                   jax.ShapeDtypeStruct((n_tiles, 8, 128), jnp.float32)),
        grid=(n_tiles,),
        in_specs=[pl.BlockSpec((1, tr, 1), lambda i: (i, 0, 0)),
                  pl.BlockSpec((1, tr, 1), lambda i: (i, 0, 0)),
                  pl.BlockSpec((v, v), lambda i: (0, 0)),
                  pl.BlockSpec((v, 128), lambda i: (0, 0))],
        out_specs=(pl.BlockSpec((tr, v), lambda i: (i, 0)),
                   pl.BlockSpec((1, 8, 128), lambda i: (i, 0, 0))),
        compiler_params=pltpu.CompilerParams(
            dimension_semantics=("parallel",),
            vmem_limit_bytes=60 * 1024 * 1024),
    )(tok3, tgt3, emb_bf, lse_mat)


def kernel(idx, emb, targets, *, row_tile=512):
    B, T = idx.shape
    V = emb.shape[0]
    N = B * T
    assert V % 128 == 0, "vocab assumed lane-aligned"

    tr = min(row_tile, N)
    n_tiles = -(-N // tr)
    Np = n_tiles * tr

    devs = jax.devices()
    ndev = 2 if (len(devs) >= 2 and n_tiles % 2 == 0 and Np == N) else 1

    tok = idx.reshape(N).astype(jnp.int32)
    tgt = targets.reshape(N).astype(jnp.int32)
    if Np != N:
        tok = jnp.pad(tok, (0, Np - N))
        tgt = jnp.pad(tgt, (0, Np - N))
    tok3 = tok.reshape(n_tiles, tr, 1)
    tgt3 = tgt.reshape(n_tiles, tr, 1)

    if ndev == 2:
        mesh = Mesh(np.asarray(devs[:2]), ("d",))
        body = functools.partial(_shard_body, tr=tr, n_tiles=n_tiles // 2,
                                 n_rows=Np // 2)
        logits, parts = jax.shard_map(
            body, mesh=mesh,
            in_specs=(P("d", None, None), P("d", None, None), P(None, None)),
            out_specs=(P("d", None), P("d", None, None)),
            check_vma=False,
        )(tok3, tgt3, emb)
    else:
        logits, parts = _shard_body(tok3, tgt3, emb,
                                    tr=tr, n_tiles=n_tiles, n_rows=N)

    loss = jnp.sum(parts[:, 0, 0]) / N
    return logits[:N], loss
